# baseline (speedup 1.0000x reference)
import functools

import jax
import jax.numpy as jnp
import numpy as np
from jax.experimental import pallas as pl
from jax.experimental.pallas import tpu as pltpu

N_NODES = 10000
B = 4


def _mm_kernel(x_ref, y_ref, o_ref, acc_ref, *, k_valid, bk):
    k = pl.program_id(2)

    @pl.when(k == 0)
    def _():
        acc_ref[...] = jnp.zeros_like(acc_ref)

    xb = x_ref[...]
    col = k * bk + jax.lax.broadcasted_iota(jnp.int32, xb.shape, 1)
    xb = jnp.where(col < k_valid, xb, 0.0)
    acc_ref[...] += jnp.dot(
        xb.astype(jnp.bfloat16),
        y_ref[...].astype(jnp.bfloat16),
        preferred_element_type=jnp.float32,
    )

    @pl.when(k == pl.num_programs(2) - 1)
    def _():
        o_ref[...] = acc_ref[...]


def _matmul_bf16(x, y, bm, bn, bk, k_valid=None):
    M, Kx = x.shape
    K, N = y.shape
    if k_valid is None:
        k_valid = Kx
    grid = (pl.cdiv(M, bm), pl.cdiv(N, bn), K // bk)
    return pl.pallas_call(
        functools.partial(_mm_kernel, k_valid=k_valid, bk=bk),
        grid=grid,
        in_specs=[
            pl.BlockSpec((bm, bk), lambda m, n, k: (m, k)),
            pl.BlockSpec((bk, bn), lambda m, n, k: (k, n)),
        ],
        out_specs=pl.BlockSpec((bm, bn), lambda m, n, k: (m, n)),
        out_shape=jax.ShapeDtypeStruct((M, N), jnp.float32),
        scratch_shapes=[pltpu.VMEM((bm, bn), jnp.float32)],
    )(x, y)


def _sage_conv(x, src, dst, Wl, Wr, b):
    msg = x[src]
    agg = jax.ops.segment_sum(msg, dst, num_segments=N_NODES)
    cnt = jax.ops.segment_sum(jnp.ones((src.shape[0],), x.dtype), dst,
                              num_segments=N_NODES)
    mean = agg / jnp.maximum(cnt, 1.0)[:, None]
    return mean @ Wl.T + x @ Wr.T + b


def kernel(input_ids, is_node, graph_x, graph_edge_index, graph_batch,
           embed_tokens, Wl1, Wr1, b1, Wl2, Wr2, b2, Wmap, bmap,
           Wq, bq, Wk, bk, Wv, bv, Wo, bo):
    SEQ = input_ids.shape[1]
    D_MODEL = embed_tokens.shape[1]
    H, DK = 16, 128
    NUM_TOKEN = (Wo.shape[0]) // D_MODEL

    src, dst = graph_edge_index[0], graph_edge_index[1]
    h = _sage_conv(graph_x, src, dst, Wl1, Wr1, b1)
    h = jax.nn.relu(h)
    h = _sage_conv(h, src, dst, Wl2, Wr2, b2)
    gsum = jax.ops.segment_sum(h, graph_batch, num_segments=B)
    gcnt = jax.ops.segment_sum(jnp.ones((N_NODES,), h.dtype), graph_batch,
                               num_segments=B)
    h_graph = gsum / jnp.maximum(gcnt, 1.0)[:, None]

    source_emb = _matmul_bf16(Wmap, embed_tokens, 1024, 2048, 640,
                              k_valid=Wmap.shape[1]) + bmap[:, None]

    q = (h_graph @ Wq.T + bq).reshape(B, 1, H, DK)
    k = (source_emb @ Wk.T + bk).reshape(-1, H, DK)
    v = (source_emb @ Wv.T + bv).reshape(-1, H, DK)
    scores = jnp.einsum('blhe,she->bhls', q, k) * (1.0 / np.sqrt(DK))
    A = jax.nn.softmax(scores, axis=-1)
    rep = jnp.einsum('bhls,she->blhe', A, v).reshape(B, 1, H * DK)
    out = rep @ Wo.T + bo
    node_embeddings = out.reshape(-1, D_MODEL)

    inputs_embeds = embed_tokens[input_ids]
    flat = inputs_embeds.reshape(B * SEQ, D_MODEL)
    mflat = is_node.reshape(-1)
    pos = jnp.cumsum(mflat.astype(jnp.int32)) - 1
    gathered = node_embeddings[jnp.clip(pos, 0, node_embeddings.shape[0] - 1)]
    flat = jnp.where(mflat[:, None], gathered, flat)
    return flat.reshape(B, SEQ, D_MODEL)
